# baseline (speedup 1.0000x reference)
import functools

import jax
import jax.numpy as jnp
from jax import lax
from jax.experimental import pallas as pl
from jax.experimental.pallas import tpu as pltpu
from jax.experimental.pallas import tpu_sc as plsc

N = 10000
E = 320000
D = 128
NC = 2
NS = 16
NW = NC * NS
B = 112
IB = 128
NBUF = 3
CH = 8
NB = 96
NCH = NB // CH
E_PAD = NW * NB * B
N_ACC = 10240
ROWS_PER_TILE = N_ACC // NS
BLK = 2000

_mesh = plsc.VectorSubcoreMesh(core_axis_name="c", subcore_axis_name="s")


def _sc_degree(dsts, zeros1d):
  @functools.partial(
      pl.kernel,
      out_type=jax.ShapeDtypeStruct((NC, N_ACC), jnp.float32),
      mesh=_mesh,
      scratch_types=[
          pltpu.VMEM((NB, IB), jnp.int32),
          pltpu.VMEM((IB,), jnp.float32),
          pltpu.VMEM_SHARED((N_ACC,), jnp.float32),
      ],
  )
  def k(dsts_hbm, z1_hbm, cnt_hbm, dst_v, ones_v, cnt_sh):
    c = lax.axis_index("c")
    s = lax.axis_index("s")
    wid = c * NS + s
    pltpu.sync_copy(dsts_hbm.at[wid], dst_v)
    for i in range(IB // 16):
      ones_v[pl.ds(i * 16, 16)] = jnp.ones((16,), jnp.float32)
    pltpu.sync_copy(z1_hbm, cnt_sh.at[pl.ds(s * ROWS_PER_TILE, ROWS_PER_TILE)])
    plsc.subcore_barrier()

    def body(j, carry):
      pltpu.sync_copy(ones_v, cnt_sh.at[dst_v.at[j]], add=True)
      return carry

    lax.fori_loop(0, NB, body, 0)
    plsc.subcore_barrier()
    pltpu.sync_copy(
        cnt_sh.at[pl.ds(s * ROWS_PER_TILE, ROWS_PER_TILE)],
        cnt_hbm.at[c, pl.ds(s * ROWS_PER_TILE, ROWS_PER_TILE)],
    )

  return k(dsts, zeros1d)


def _sc_scatter(table, srcs, dsts, zeros2d):
  @functools.partial(
      pl.kernel,
      out_type=jax.ShapeDtypeStruct((NC, N_ACC, D), jnp.float32),
      mesh=_mesh,
      scratch_types=[
          [pltpu.VMEM((CH, IB), jnp.int32)] * 2,
          [pltpu.VMEM((CH, IB), jnp.int32)] * 2,
          [pltpu.SemaphoreType.DMA] * 2,
          [pltpu.SemaphoreType.DMA] * 2,
          [pltpu.VMEM((B, D), jnp.float32)] * NBUF,
          [pltpu.SemaphoreType.DMA] * NBUF,
          [pltpu.SemaphoreType.DMA] * NBUF,
          pltpu.VMEM_SHARED((N_ACC, D), jnp.float32),
      ],
  )
  def k(table_hbm, srcs_hbm, dsts_hbm, z2_hbm, acc_hbm,
        src_c, dst_c, issem, idsem, bufs, gsem, ssem, acc_sh):
    c = lax.axis_index("c")
    s = lax.axis_index("s")
    wid = c * NS + s
    rows = pl.ds(s * ROWS_PER_TILE, ROWS_PER_TILE)

    def idx_load(ch):
      sl = ch % 2
      win = pl.ds(ch * CH, CH)
      pltpu.async_copy(srcs_hbm.at[wid, win], src_c[sl], issem[sl])
      pltpu.async_copy(dsts_hbm.at[wid, win], dst_c[sl], idsem[sl])

    def idx_wait(ch):
      sl = ch % 2
      win = pl.ds(ch * CH, CH)
      pltpu.make_async_copy(srcs_hbm.at[wid, win], src_c[sl], issem[sl]).wait()
      pltpu.make_async_copy(dsts_hbm.at[wid, win], dst_c[sl], idsem[sl]).wait()

    def src_ref(j):
      return src_c[(j // CH) % 2].at[j % CH, pl.ds(0, B)]

    def dst_ref(j):
      return dst_c[(j // CH) % 2].at[j % CH, pl.ds(0, B)]

    def gather(j, t):
      pltpu.async_copy(table_hbm.at[src_ref(j)], bufs[t], gsem[t])

    def gather_wait(j, t):
      pltpu.make_async_copy(table_hbm.at[src_ref(j)], bufs[t], gsem[t]).wait()

    def scat(j, t):
      pltpu.async_copy(bufs[t], acc_sh.at[dst_ref(j)], ssem[t], add=True)

    def scat_wait(j, t):
      pltpu.make_async_copy(bufs[t], acc_sh.at[dst_ref(j)], ssem[t]).wait()

    idx_load(0)
    pltpu.sync_copy(z2_hbm, acc_sh.at[rows])
    idx_wait(0)
    for t in range(NBUF):
      gather(t, t)
    idx_load(1)
    plsc.subcore_barrier()

    for j in range(NB):
      t = j % NBUF
      gather_wait(j, t)
      scat(j, t)
      nj = j + NBUF
      if nj < NB:
        if nj % CH == 0:
          idx_wait(nj // CH)
        scat_wait(j, t)
        gather(nj, t)
      else:
        scat_wait(j, t)
      if (j + 1) % CH == 0 and (j + 1) // CH + 1 < NCH:
        idx_load((j + 1) // CH + 1)

    plsc.subcore_barrier()
    pltpu.sync_copy(acc_sh.at[rows], acc_hbm.at[c, rows])

  return k(table, srcs, dsts, zeros2d)


def _m1_body(x_r, w_r, b_r, ca_r, cb_r, yw_r, z_r):
  xw = jnp.dot(x_r[...], w_r[...], preferred_element_type=jnp.float32)
  deg = ca_r[...] + cb_r[...] + 1.0
  dis = lax.rsqrt(deg)
  yw_r[...] = dis * xw
  z_r[...] = xw * (1.0 / deg) + b_r[...]


def _m2_body(aa_r, ab_r, z1_r, ca_r, cb_r, w_r, b_r, yw_r, z2_r):
  deg = ca_r[...] + cb_r[...] + 1.0
  dis = lax.rsqrt(deg)
  h = jnp.maximum(dis * (aa_r[0] + ab_r[0]) + z1_r[...], 0.0)
  xw = jnp.dot(h, w_r[...], preferred_element_type=jnp.float32)
  yw_r[...] = dis * xw
  z2_r[...] = xw * (1.0 / deg) + b_r[...]


def _m3_body(aa_r, ab_r, z2_r, ca_r, cb_r, out_r):
  deg = ca_r[...] + cb_r[...] + 1.0
  dis = lax.rsqrt(deg)
  out_r[...] = dis * (aa_r[0] + ab_r[0]) + z2_r[...]


_row = pl.BlockSpec((BLK, D), lambda i: (i, 0))
_col = pl.BlockSpec((BLK, 1), lambda i: (i, 0))
_wsp = pl.BlockSpec((D, D), lambda i: (0, 0))
_bsp = pl.BlockSpec((1, D), lambda i: (0, 0))
_acc_a = pl.BlockSpec((1, BLK, D), lambda i: (0, i, 0))
_acc_b = pl.BlockSpec((1, BLK, D), lambda i: (1, i, 0))
_G = (N // BLK,)
_OUT2 = (
    jax.ShapeDtypeStruct((N, D), jnp.float32),
    jax.ShapeDtypeStruct((N, D), jnp.float32),
)

_m1 = pl.pallas_call(
    _m1_body, grid=_G,
    in_specs=[_row, _wsp, _bsp, _col, _col],
    out_specs=(_row, _row), out_shape=_OUT2)

_m2 = pl.pallas_call(
    _m2_body, grid=_G,
    in_specs=[_acc_a, _acc_b, _row, _col, _col, _wsp, _bsp],
    out_specs=(_row, _row), out_shape=_OUT2)

_m3 = pl.pallas_call(
    _m3_body, grid=_G,
    in_specs=[_acc_a, _acc_b, _row, _col, _col],
    out_specs=_row, out_shape=jax.ShapeDtypeStruct((N, D), jnp.float32))


@jax.jit
def _run(x, edge_index, W1, b1, W2, b2):
  src = edge_index[0].astype(jnp.int32)
  dst = edge_index[1].astype(jnp.int32)
  pad = E_PAD - E
  pad_src = (jnp.arange(pad, dtype=jnp.int32) * 37) % N
  pad_dst = N + (jnp.arange(pad, dtype=jnp.int32) % (N_ACC - N))
  srcs = jnp.pad(
      jnp.concatenate([src, pad_src]).reshape(NW, NB, B),
      ((0, 0), (0, 0), (0, IB - B)))
  dsts = jnp.pad(
      jnp.concatenate([dst, pad_dst]).reshape(NW, NB, B),
      ((0, 0), (0, 0), (0, IB - B)), constant_values=N)
  zeros1d = jnp.zeros((ROWS_PER_TILE,), jnp.float32)
  zeros2d = jnp.zeros((ROWS_PER_TILE, D), jnp.float32)

  cnt = _sc_degree(dsts, zeros1d)
  ca = cnt[0, :N].reshape(N, 1)
  cb = cnt[1, :N].reshape(N, 1)
  b1r = b1.reshape(1, D)
  b2r = b2.reshape(1, D)

  yw1, z1 = _m1(x, W1, b1r, ca, cb)
  acc1 = _sc_scatter(yw1, srcs, dsts, zeros2d)
  yw2, z2 = _m2(acc1, acc1, z1, ca, cb, W2, b2r)
  acc2 = _sc_scatter(yw2, srcs, dsts, zeros2d)
  return _m3(acc2, acc2, z2, ca, cb)


def kernel(x, edge_index, W1, b1, W2, b2):
  return _run(x, edge_index, W1, b1, W2, b2)


# speedup vs baseline: 1.0410x; 1.0410x over previous
import functools

import jax
import jax.numpy as jnp
from jax import lax
from jax.experimental import pallas as pl
from jax.experimental.pallas import tpu as pltpu
from jax.experimental.pallas import tpu_sc as plsc

N = 10000
E = 320000
D = 128
NC = 2
NS = 16
NW = NC * NS
B = 128
NBUF = 2
CH = 16
NB = -(-E // (NW * B * NBUF)) * NBUF
NCH = NB // CH
E_PAD = NW * NB * B
N_ACC = 10240
ROWS_PER_TILE = N_ACC // NS
BLK = 2000

_mesh = plsc.VectorSubcoreMesh(core_axis_name="c", subcore_axis_name="s")


def _sc_degree(dsts, zeros1d):
  @functools.partial(
      pl.kernel,
      out_type=jax.ShapeDtypeStruct((NC, N_ACC), jnp.float32),
      mesh=_mesh,
      scratch_types=[
          pltpu.VMEM((NB, B), jnp.int32),
          pltpu.VMEM((B,), jnp.float32),
          pltpu.VMEM_SHARED((N_ACC,), jnp.float32),
      ],
  )
  def k(dsts_hbm, z1_hbm, cnt_hbm, dst_v, ones_v, cnt_sh):
    c = lax.axis_index("c")
    s = lax.axis_index("s")
    wid = c * NS + s
    pltpu.sync_copy(dsts_hbm.at[wid], dst_v)
    for i in range(B // 16):
      ones_v[pl.ds(i * 16, 16)] = jnp.ones((16,), jnp.float32)
    pltpu.sync_copy(z1_hbm, cnt_sh.at[pl.ds(s * ROWS_PER_TILE, ROWS_PER_TILE)])
    plsc.subcore_barrier()

    def body(j, carry):
      pltpu.sync_copy(ones_v, cnt_sh.at[dst_v.at[j]], add=True)
      return carry

    lax.fori_loop(0, NB, body, 0)
    plsc.subcore_barrier()
    pltpu.sync_copy(
        cnt_sh.at[pl.ds(s * ROWS_PER_TILE, ROWS_PER_TILE)],
        cnt_hbm.at[c, pl.ds(s * ROWS_PER_TILE, ROWS_PER_TILE)],
    )

  return k(dsts, zeros1d)


def _sc_scatter(table, srcs, dsts, zblk):
  @functools.partial(
      pl.kernel,
      out_type=jax.ShapeDtypeStruct((NC, N_ACC, D), jnp.float32),
      mesh=_mesh,
      scratch_types=[
          [pltpu.VMEM((CH, B), jnp.int32)] * 2,
          [pltpu.VMEM((CH, B), jnp.int32)] * 2,
          [pltpu.SemaphoreType.DMA] * 2,
          [pltpu.SemaphoreType.DMA] * 2,
          [pltpu.VMEM((B, D), jnp.float32)] * NBUF,
          [pltpu.SemaphoreType.DMA] * NBUF,
          [pltpu.SemaphoreType.DMA] * NBUF,
          pltpu.VMEM_SHARED((N_ACC, D), jnp.float32),
      ],
  )
  def k(table_hbm, srcs_hbm, dsts_hbm, zb_hbm, acc_hbm,
        src_c, dst_c, issem, idsem, bufs, gsem, ssem, acc_sh):
    c = lax.axis_index("c")
    s = lax.axis_index("s")
    wid = c * NS + s
    rows = pl.ds(s * ROWS_PER_TILE, ROWS_PER_TILE)

    def idx_load(ch):
      sl = ch % 2
      win = pl.ds(ch * CH, CH)
      pltpu.async_copy(srcs_hbm.at[wid, win], src_c[sl], issem[sl])
      pltpu.async_copy(dsts_hbm.at[wid, win], dst_c[sl], idsem[sl])

    def idx_wait(ch):
      sl = ch % 2
      win = pl.ds(ch * CH, CH)
      pltpu.make_async_copy(srcs_hbm.at[wid, win], src_c[sl], issem[sl]).wait()
      pltpu.make_async_copy(dsts_hbm.at[wid, win], dst_c[sl], idsem[sl]).wait()

    def src_ref(j):
      return src_c[(j // CH) % 2].at[j % CH]

    def dst_ref(j):
      return dst_c[(j // CH) % 2].at[j % CH]

    def gather(j, t):
      pltpu.async_copy(table_hbm.at[src_ref(j)], bufs[t], gsem[t])

    def gather_wait(j, t):
      pltpu.make_async_copy(table_hbm.at[src_ref(j)], bufs[t], gsem[t]).wait()

    def scat(j, t):
      pltpu.async_copy(bufs[t], acc_sh.at[dst_ref(j)], ssem[t], add=True)

    def scat_wait(j, t):
      pltpu.make_async_copy(bufs[t], acc_sh.at[dst_ref(j)], ssem[t]).wait()

    idx_load(0)
    pltpu.sync_copy(zb_hbm, bufs[0])
    for r in range(ROWS_PER_TILE // B):
      pltpu.sync_copy(
          bufs[0], acc_sh.at[pl.ds(s * ROWS_PER_TILE + r * B, B)])
    idx_wait(0)
    for t in range(NBUF):
      gather(t, t)
    idx_load(1)
    plsc.subcore_barrier()

    for j in range(NB):
      t = j % NBUF
      gather_wait(j, t)
      scat(j, t)
      nj = j + NBUF
      if nj < NB:
        if nj % CH == 0:
          idx_wait(nj // CH)
        scat_wait(j, t)
        gather(nj, t)
      else:
        scat_wait(j, t)
      if (j + 1) % CH == 0 and (j + 1) // CH + 1 < NCH:
        idx_load((j + 1) // CH + 1)

    plsc.subcore_barrier()
    pltpu.sync_copy(acc_sh.at[rows], acc_hbm.at[c, rows])

  return k(table, srcs, dsts, zblk)


def _m1_body(x_r, w_r, ca_r, cb_r, yw_r):
  xw = jnp.dot(x_r[...], w_r[...], preferred_element_type=jnp.float32)
  dis = lax.rsqrt(ca_r[...] + cb_r[...] + 1.0)
  yw_r[...] = dis * xw


def _m2_body(aa_r, ab_r, yw1_r, ca_r, cb_r, w_r, b_r, yw_r):
  dis = lax.rsqrt(ca_r[...] + cb_r[...] + 1.0)
  h = jnp.maximum(dis * (aa_r[0] + ab_r[0] + yw1_r[...]) + b_r[...], 0.0)
  xw = jnp.dot(h, w_r[...], preferred_element_type=jnp.float32)
  yw_r[...] = dis * xw


def _m3_body(aa_r, ab_r, yw2_r, ca_r, cb_r, b_r, out_r):
  dis = lax.rsqrt(ca_r[...] + cb_r[...] + 1.0)
  out_r[...] = dis * (aa_r[0] + ab_r[0] + yw2_r[...]) + b_r[...]


_row = pl.BlockSpec((BLK, D), lambda i: (i, 0))
_col = pl.BlockSpec((BLK, 1), lambda i: (i, 0))
_wsp = pl.BlockSpec((D, D), lambda i: (0, 0))
_bsp = pl.BlockSpec((1, D), lambda i: (0, 0))
_acc_a = pl.BlockSpec((1, BLK, D), lambda i: (0, i, 0))
_acc_b = pl.BlockSpec((1, BLK, D), lambda i: (1, i, 0))
_G = (N // BLK,)
_OUT1 = jax.ShapeDtypeStruct((N, D), jnp.float32)

_m1 = pl.pallas_call(
    _m1_body, grid=_G,
    in_specs=[_row, _wsp, _col, _col],
    out_specs=_row, out_shape=_OUT1)

_m2 = pl.pallas_call(
    _m2_body, grid=_G,
    in_specs=[_acc_a, _acc_b, _row, _col, _col, _wsp, _bsp],
    out_specs=_row, out_shape=_OUT1)

_m3 = pl.pallas_call(
    _m3_body, grid=_G,
    in_specs=[_acc_a, _acc_b, _row, _col, _col, _bsp],
    out_specs=_row, out_shape=_OUT1)


@jax.jit
def _run(x, edge_index, W1, b1, W2, b2):
  src = edge_index[0].astype(jnp.int32)
  dst = edge_index[1].astype(jnp.int32)
  pad = E_PAD - E
  pad_src = (jnp.arange(pad, dtype=jnp.int32) * 37) % N
  pad_dst = N + (jnp.arange(pad, dtype=jnp.int32) % (N_ACC - N))
  srcs = jnp.concatenate([src, pad_src]).reshape(NW, NB, B)
  dsts = jnp.concatenate([dst, pad_dst]).reshape(NW, NB, B)
  zeros1d = jnp.zeros((ROWS_PER_TILE,), jnp.float32)
  zblk = jnp.zeros((B, D), jnp.float32)

  cnt = _sc_degree(dsts, zeros1d)
  ca = cnt[0, :N].reshape(N, 1)
  cb = cnt[1, :N].reshape(N, 1)
  b1r = b1.reshape(1, D)
  b2r = b2.reshape(1, D)

  yw1 = _m1(x, W1, ca, cb)
  acc1 = _sc_scatter(yw1, srcs, dsts, zblk)
  yw2 = _m2(acc1, acc1, yw1, ca, cb, W2, b1r)
  acc2 = _sc_scatter(yw2, srcs, dsts, zblk)
  return _m3(acc2, acc2, yw2, ca, cb, b2r)


def kernel(x, edge_index, W1, b1, W2, b2):
  return _run(x, edge_index, W1, b1, W2, b2)
